# Initial kernel scaffold; baseline (speedup 1.0000x reference)
import jax
import jax.numpy as jnp
from jax.experimental import pallas as pl


def kernel(inputs, embeddings):
    raise NotImplementedError("write your pallas kernel here")


# speedup vs baseline: 1.1051x; 1.1051x over previous
import functools

import jax
import jax.numpy as jnp
from jax import lax
from jax.experimental import pallas as pl
from jax.experimental.pallas import tpu as pltpu
from jax.experimental.pallas import tpu_sc as plsc

VOCAB = 1000000
EMBED_DIM = 32
BATCH = 16384
HIST = 50

B = BATCH * HIST
NW = 32
BPW = B // NW
IPG = 128
G = BPW // IPG
K = 10
CK = K * IPG
NSC = G // K

_mesh = plsc.VectorSubcoreMesh(core_axis_name="c", subcore_axis_name="s")


@functools.partial(
    pl.kernel,
    mesh=_mesh,
    out_type=jax.ShapeDtypeStruct((B, EMBED_DIM), jnp.float32),
    scratch_types=[
        pltpu.VMEM((G, IPG), jnp.int32),
        pltpu.VMEM((CK, EMBED_DIM), jnp.float32),
        pltpu.SemaphoreType.DMA,
    ],
    compiler_params=pltpu.CompilerParams(use_tc_tiling_on_sc=False),
)
def _gather_kernel(table_hbm, idx_hbm, out_hbm, idx_v, rows_v, sem):
  wid = lax.axis_index("s") * 2 + lax.axis_index("c")
  base = wid * BPW
  pltpu.sync_copy(idx_hbm.at[wid], idx_v)

  def super_chunk(g, _):
    copies = []
    for j in range(K):
      copies.append(
          pltpu.async_copy(
              table_hbm.at[idx_v.at[g * K + j]],
              rows_v.at[pl.ds(j * IPG, IPG)],
              sem,
          )
      )
    for cp in copies:
      cp.wait()
    pltpu.sync_copy(rows_v, out_hbm.at[pl.ds(base + g * CK, CK)])
    return 0

  lax.fori_loop(0, NSC, super_chunk, 0)


def kernel(inputs, embeddings):
  idx = inputs.astype(jnp.int32).reshape(NW, G, IPG)
  out = _gather_kernel(embeddings, idx)
  return out.reshape(BATCH, HIST, EMBED_DIM)
